# Initial kernel scaffold; baseline (speedup 1.0000x reference)
import jax
import jax.numpy as jnp
from jax.experimental import pallas as pl


def kernel(features, adj, weight):
    raise NotImplementedError("write your pallas kernel here")


# speedup vs baseline: 1.0056x; 1.0056x over previous
import math

import jax
import jax.numpy as jnp
from jax.experimental import pallas as pl
from jax.experimental.pallas import tpu as pltpu


def _xw_body(x_ref, w_ref, o_ref):
    o_ref[...] = jnp.dot(x_ref[...], w_ref[...],
                         preferred_element_type=jnp.float32)


def _spmm_body(adj_ref, sup_ref, out_ref):
    acc = jnp.dot(adj_ref[...], sup_ref[...],
                  preferred_element_type=jnp.float32)
    out_ref[...] = jnp.maximum(acc, 0.0)


def kernel(features, adj, weight):
    n, d_in = features.shape
    d_out = weight.shape[1]

    bmx = math.gcd(n, 2000)
    support = pl.pallas_call(
        _xw_body,
        grid=(n // bmx,),
        in_specs=[
            pl.BlockSpec((bmx, d_in), lambda m: (m, 0)),
            pl.BlockSpec((d_in, d_out), lambda m: (0, 0)),
        ],
        out_specs=pl.BlockSpec((bmx, d_out), lambda m: (m, 0)),
        out_shape=jax.ShapeDtypeStruct((n, d_out), jnp.float32),
        compiler_params=pltpu.CompilerParams(
            dimension_semantics=("parallel",),
        ),
    )(features, weight)

    bm = math.gcd(n, 400)
    out = pl.pallas_call(
        _spmm_body,
        grid=(n // bm,),
        in_specs=[
            pl.BlockSpec((bm, n), lambda m: (m, 0)),
            pl.BlockSpec((n, d_out), lambda m: (0, 0)),
        ],
        out_specs=pl.BlockSpec((bm, d_out), lambda m: (m, 0)),
        out_shape=jax.ShapeDtypeStruct((n, d_out), jnp.float32),
        compiler_params=pltpu.CompilerParams(
            dimension_semantics=("parallel",),
        ),
    )(adj, support)
    return out


# speedup vs baseline: 1.0061x; 1.0005x over previous
import math

import jax
import jax.numpy as jnp
from jax.experimental import pallas as pl
from jax.experimental.pallas import tpu as pltpu


def _xw_body(x_ref, w_ref, o_ref):
    o_ref[...] = jnp.dot(x_ref[...], w_ref[...],
                         preferred_element_type=jnp.float32)


def _spmm_body(adj_ref, sup_ref, out_ref):
    acc = jnp.dot(adj_ref[...].astype(jnp.bfloat16),
                  sup_ref[...].astype(jnp.bfloat16),
                  preferred_element_type=jnp.float32)
    out_ref[...] = jnp.maximum(acc, 0.0)


def kernel(features, adj, weight):
    n, d_in = features.shape
    d_out = weight.shape[1]

    bmx = math.gcd(n, 2000)
    support = pl.pallas_call(
        _xw_body,
        grid=(n // bmx,),
        in_specs=[
            pl.BlockSpec((bmx, d_in), lambda m: (m, 0)),
            pl.BlockSpec((d_in, d_out), lambda m: (0, 0)),
        ],
        out_specs=pl.BlockSpec((bmx, d_out), lambda m: (m, 0)),
        out_shape=jax.ShapeDtypeStruct((n, d_out), jnp.float32),
        compiler_params=pltpu.CompilerParams(
            dimension_semantics=("parallel",),
        ),
    )(features, weight)

    bm = math.gcd(n, 400)
    out = pl.pallas_call(
        _spmm_body,
        grid=(n // bm,),
        in_specs=[
            pl.BlockSpec((bm, n), lambda m: (m, 0)),
            pl.BlockSpec((n, d_out), lambda m: (0, 0)),
        ],
        out_specs=pl.BlockSpec((bm, d_out), lambda m: (m, 0)),
        out_shape=jax.ShapeDtypeStruct((n, d_out), jnp.float32),
        compiler_params=pltpu.CompilerParams(
            dimension_semantics=("parallel",),
        ),
    )(adj, support)
    return out


# speedup vs baseline: 1.0229x; 1.0167x over previous
import math

import jax
import jax.numpy as jnp
from jax.experimental import pallas as pl
from jax.experimental.pallas import tpu as pltpu


def _xw_body(x_ref, w_ref, o_ref):
    o_ref[...] = jnp.dot(x_ref[...], w_ref[...],
                         preferred_element_type=jnp.float32
                         ).astype(jnp.bfloat16)


def _spmm_body(bm, adj_a_ref, adj_b_ref, sup_ref, out_ref):
    sup = sup_ref[...]
    acc_a = jnp.dot(adj_a_ref[...].astype(jnp.bfloat16), sup,
                    preferred_element_type=jnp.float32)
    out_ref[pl.ds(0, bm), :] = jnp.maximum(acc_a, 0.0)
    acc_b = jnp.dot(adj_b_ref[...].astype(jnp.bfloat16), sup,
                    preferred_element_type=jnp.float32)
    out_ref[pl.ds(bm, bm), :] = jnp.maximum(acc_b, 0.0)


def kernel(features, adj, weight):
    n, d_in = features.shape
    d_out = weight.shape[1]

    bmx = math.gcd(n, 2000)
    support = pl.pallas_call(
        _xw_body,
        grid=(n // bmx,),
        in_specs=[
            pl.BlockSpec((bmx, d_in), lambda m: (m, 0)),
            pl.BlockSpec((d_in, d_out), lambda m: (0, 0)),
        ],
        out_specs=pl.BlockSpec((bmx, d_out), lambda m: (m, 0)),
        out_shape=jax.ShapeDtypeStruct((n, d_out), jnp.bfloat16),
        compiler_params=pltpu.CompilerParams(
            dimension_semantics=("parallel",),
        ),
    )(features, weight)

    bm = math.gcd(n, 200)
    grid = (n // (2 * bm),)
    out = pl.pallas_call(
        lambda *refs: _spmm_body(bm, *refs),
        grid=grid,
        in_specs=[
            pl.BlockSpec((bm, n), lambda m: (2 * m, 0)),
            pl.BlockSpec((bm, n), lambda m: (2 * m + 1, 0)),
            pl.BlockSpec((n, d_out), lambda m: (0, 0)),
        ],
        out_specs=pl.BlockSpec((2 * bm, d_out), lambda m: (m, 0)),
        out_shape=jax.ShapeDtypeStruct((n, d_out), jnp.float32),
        compiler_params=pltpu.CompilerParams(
            dimension_semantics=("parallel",),
        ),
    )(adj, adj, support)
    return out


# speedup vs baseline: 1.0691x; 1.0452x over previous
import functools
import math

import jax
import jax.numpy as jnp
from jax.experimental import pallas as pl
from jax.experimental.pallas import tpu as pltpu


def _fused_body(g1, bf, bm, x_ref, w_ref, adj_a_ref, adj_b_ref, out_ref,
                sup_ref):
    g = pl.program_id(0)

    @pl.when(g < g1)
    def _support_phase():
        sup = jnp.dot(x_ref[...], w_ref[...],
                      preferred_element_type=jnp.float32)
        sup_ref[pl.ds(g * bf, bf), :] = sup.astype(jnp.bfloat16)

    @pl.when(g >= g1)
    def _spmm_phase():
        sup = sup_ref[...]
        acc_a = jnp.dot(adj_a_ref[...].astype(jnp.bfloat16), sup,
                        preferred_element_type=jnp.float32)
        out_ref[pl.ds(0, bm), :] = jnp.maximum(acc_a, 0.0)
        acc_b = jnp.dot(adj_b_ref[...].astype(jnp.bfloat16), sup,
                        preferred_element_type=jnp.float32)
        out_ref[pl.ds(bm, bm), :] = jnp.maximum(acc_b, 0.0)


def kernel(features, adj, weight):
    n, d_in = features.shape
    d_out = weight.shape[1]

    bf = math.gcd(n, 2000)
    bm = math.gcd(n, 200)
    g1 = n // bf
    g2 = n // (2 * bm)

    body = functools.partial(_fused_body, g1, bf, bm)
    out = pl.pallas_call(
        body,
        grid=(g1 + g2,),
        in_specs=[
            pl.BlockSpec((bf, d_in),
                         lambda g: (jnp.minimum(g, g1 - 1), 0)),
            pl.BlockSpec((d_in, d_out), lambda g: (0, 0)),
            pl.BlockSpec((bm, n),
                         lambda g: (2 * jnp.maximum(g - g1, 0), 0)),
            pl.BlockSpec((bm, n),
                         lambda g: (2 * jnp.maximum(g - g1, 0) + 1, 0)),
        ],
        out_specs=pl.BlockSpec((2 * bm, d_out),
                               lambda g: (jnp.maximum(g - g1, 0), 0)),
        out_shape=jax.ShapeDtypeStruct((n, d_out), jnp.float32),
        scratch_shapes=[pltpu.VMEM((n, d_out), jnp.bfloat16)],
        compiler_params=pltpu.CompilerParams(
            dimension_semantics=("arbitrary",),
        ),
    )(features, weight, adj, adj)
    return out


# speedup vs baseline: 1.1055x; 1.0340x over previous
import functools
import math

import jax
import jax.numpy as jnp
from jax.experimental import pallas as pl
from jax.experimental.pallas import tpu as pltpu

_NBUF = 3


def _copy_panel(adj_ref, bufs_ref, sems_ref, panel, slot, bm):
    pltpu.make_async_copy(
        adj_ref.at[pl.ds(panel * bm, bm), :],
        bufs_ref.at[slot],
        sems_ref.at[slot],
    ).start()


def _fused_body(g1, g2, bf, bm, x_ref, w_ref, adj_ref, out_ref,
                sup_ref, bufs_ref, sems_ref):
    g = pl.program_id(0)

    @pl.when(g == 0)
    def _prime_dma():
        for i in range(min(_NBUF, g2)):
            _copy_panel(adj_ref, bufs_ref, sems_ref, i, i, bm)

    @pl.when(g < g1)
    def _support_phase():
        sup = jnp.dot(x_ref[...], w_ref[...],
                      preferred_element_type=jnp.float32)
        sup_ref[pl.ds(g * bf, bf), :] = sup.astype(jnp.bfloat16)

    @pl.when(g >= g1)
    def _spmm_phase():
        m = g - g1
        slot = jax.lax.rem(m, _NBUF)
        pltpu.make_async_copy(
            adj_ref.at[pl.ds(m * bm, bm), :],
            bufs_ref.at[slot],
            sems_ref.at[slot],
        ).wait()
        acc = jnp.dot(bufs_ref[slot].astype(jnp.bfloat16), sup_ref[...],
                      preferred_element_type=jnp.float32)
        out_ref[...] = jnp.maximum(acc, 0.0)

        @pl.when(m + _NBUF < g2)
        def _refill():
            _copy_panel(adj_ref, bufs_ref, sems_ref, m + _NBUF, slot, bm)


def kernel(features, adj, weight):
    n, d_in = features.shape
    d_out = weight.shape[1]

    bf = math.gcd(n, 2000)
    bm = math.gcd(n, 200)
    g1 = n // bf
    g2 = n // bm

    body = functools.partial(_fused_body, g1, g2, bf, bm)
    out = pl.pallas_call(
        body,
        grid=(g1 + g2,),
        in_specs=[
            pl.BlockSpec((bf, d_in),
                         lambda g: (jnp.minimum(g, g1 - 1), 0)),
            pl.BlockSpec((d_in, d_out), lambda g: (0, 0)),
            pl.BlockSpec(memory_space=pltpu.MemorySpace.HBM),
        ],
        out_specs=pl.BlockSpec((bm, d_out),
                               lambda g: (jnp.maximum(g - g1, 0), 0)),
        out_shape=jax.ShapeDtypeStruct((n, d_out), jnp.float32),
        scratch_shapes=[
            pltpu.VMEM((n, d_out), jnp.bfloat16),
            pltpu.VMEM((_NBUF, bm, n), jnp.float32),
            pltpu.SemaphoreType.DMA((_NBUF,)),
        ],
        compiler_params=pltpu.CompilerParams(
            dimension_semantics=("arbitrary",),
        ),
    )(features, weight, adj)
    return out
